# baseline (device time: 52689 ns/iter reference)
import jax
import jax.numpy as jnp
from jax import lax
from jax.experimental import pallas as pl
from jax.experimental.pallas import tpu as pltpu

N_DEV = 4
B, SQ, SKV, DH = 2, 256, 256, 64
H_PER = 4
HD = H_PER * DH
D_MODEL = 512


def kernel(x, Wq, K_ext, V_ext, Wo):
    K2 = K_ext.reshape(B, SKV, HD)
    V2 = V_ext.reshape(B, SKV, HD)

    def body(x_ref, wq_ref, k_ref, v_ref, wo_ref, out_ref,
             comm_ref, send_sems, recv_sems):
        my = lax.axis_index("i")
        left = lax.rem(my + N_DEV - 1, N_DEV)
        right = lax.rem(my + 1, N_DEV)

        barrier_sem = pltpu.get_barrier_semaphore()
        for nbr in (left, right):
            pl.semaphore_signal(
                barrier_sem, inc=1,
                device_id=(nbr,), device_id_type=pl.DeviceIdType.MESH,
            )
        pl.semaphore_wait(barrier_sem, 2)

        wq = wq_ref[:, pl.ds(my * HD, HD)]
        wo = wo_ref[pl.ds(my * HD, HD), :]

        qb = lax.broadcasted_iota(jnp.int32, (SQ, SKV), 0) // 64
        kb = lax.broadcasted_iota(jnp.int32, (SQ, SKV), 1) // 64
        mask = (qb == kb) | ((kb % 4) == (qb % 4))

        for b in range(B):
            q = jnp.dot(x_ref[b], wq, preferred_element_type=jnp.float32)
            kbat = k_ref[b]
            vbat = v_ref[b]
            ctx_parts = []
            for h in range(H_PER):
                qh = q[:, h * DH:(h + 1) * DH]
                kh = kbat[:, h * DH:(h + 1) * DH]
                vh = vbat[:, h * DH:(h + 1) * DH]
                s = lax.dot_general(
                    qh, kh, (((1,), (1,)), ((), ())),
                    preferred_element_type=jnp.float32,
                ) * 0.125
                s = jnp.where(mask, s, -1e9)
                s = s - jnp.max(s, axis=-1, keepdims=True)
                w = jnp.exp(s)
                w = w / jnp.sum(w, axis=-1, keepdims=True)
                ctx_parts.append(
                    jnp.dot(w, vh, preferred_element_type=jnp.float32))
            ctx = jnp.concatenate(ctx_parts, axis=1)
            part = jnp.dot(ctx, wo, preferred_element_type=jnp.float32)
            out_ref[b] = part
            comm_ref[0, b] = part

        for h in range(N_DEV - 1):
            rdma = pltpu.make_async_remote_copy(
                src_ref=comm_ref.at[h],
                dst_ref=comm_ref.at[h + 1],
                send_sem=send_sems.at[h],
                recv_sem=recv_sems.at[h],
                device_id=(right,),
                device_id_type=pl.DeviceIdType.MESH,
            )
            rdma.start()
            rdma.wait()
            for b in range(B):
                out_ref[b] = out_ref[b] + comm_ref[h + 1, b]

    return pl.pallas_call(
        body,
        out_shape=jax.ShapeDtypeStruct((B, SQ, D_MODEL), jnp.float32),
        in_specs=[pl.BlockSpec(memory_space=pltpu.VMEM)] * 5,
        out_specs=pl.BlockSpec(memory_space=pltpu.VMEM),
        scratch_shapes=[
            pltpu.VMEM((N_DEV, B, SQ, D_MODEL), jnp.float32),
            pltpu.SemaphoreType.DMA((N_DEV - 1,)),
            pltpu.SemaphoreType.DMA((N_DEV - 1,)),
        ],
        compiler_params=pltpu.CompilerParams(collective_id=0),
    )(x, Wq, K2, V2, Wo)


# device time: 24962 ns/iter; 2.1108x vs baseline; 2.1108x over previous
import jax
import jax.numpy as jnp
from jax import lax
from jax.experimental import pallas as pl
from jax.experimental.pallas import tpu as pltpu

N_DEV = 4
B, SQ, SKV, DH = 2, 256, 256, 64
H_PER = 4
HD = H_PER * DH
D_MODEL = 512
ROWS = B * SQ
QR = ROWS // N_DEV


def kernel(x, Wq, K_ext, V_ext, Wo):
    x2 = x.reshape(ROWS, D_MODEL)
    K2 = K_ext.reshape(B, SKV, HD)
    V2 = V_ext.reshape(B, SKV, HD)

    def body(x_ref, wq_ref, k_ref, v_ref, wo_ref, out_ref,
             ctx_ref, rs_ref, rs_send, rs_recv, ag_send, ag_recv):
        my = lax.axis_index("i")

        barrier_sem = pltpu.get_barrier_semaphore()
        for r in range(1, N_DEV):
            pl.semaphore_signal(
                barrier_sem, inc=1,
                device_id=(lax.rem(my + r, N_DEV),),
                device_id_type=pl.DeviceIdType.MESH,
            )
        pl.semaphore_wait(barrier_sem, N_DEV - 1)

        wq = wq_ref[:, pl.ds(my * HD, HD)]
        q_all = jnp.dot(x_ref[...], wq, preferred_element_type=jnp.float32)

        qb = lax.broadcasted_iota(jnp.int32, (SQ, SKV), 0) // 64
        kb = lax.broadcasted_iota(jnp.int32, (SQ, SKV), 1) // 64
        mask = (qb == kb) | ((kb % 4) == (qb % 4))

        for b in range(B):
            q = q_all[b * SQ:(b + 1) * SQ, :]
            kbat = k_ref[b]
            vbat = v_ref[b]
            ctx_parts = []
            for h in range(H_PER):
                qh = q[:, h * DH:(h + 1) * DH]
                kh = kbat[:, h * DH:(h + 1) * DH]
                vh = vbat[:, h * DH:(h + 1) * DH]
                s = lax.dot_general(
                    qh, kh, (((1,), (1,)), ((), ())),
                    preferred_element_type=jnp.float32,
                ) * 0.125
                s = jnp.where(mask, s, -1e9)
                s = s - jnp.max(s, axis=-1, keepdims=True)
                w = jnp.exp(s)
                w = w / jnp.sum(w, axis=-1, keepdims=True)
                ctx_parts.append(
                    jnp.dot(w, vh, preferred_element_type=jnp.float32))
            ctx_ref[pl.ds(b * SQ, SQ)] = jnp.concatenate(ctx_parts, axis=1)

        a_rdmas = []
        for r in range(1, N_DEV):
            tgt = lax.rem(my + r, N_DEV)
            rdma = pltpu.make_async_remote_copy(
                src_ref=ctx_ref.at[pl.ds(tgt * QR, QR)],
                dst_ref=rs_ref.at[r],
                send_sem=rs_send.at[r],
                recv_sem=rs_recv.at[r],
                device_id=(tgt,),
                device_id_type=pl.DeviceIdType.MESH,
            )
            rdma.start()
            a_rdmas.append(rdma)

        wo_mine = wo_ref[pl.ds(my * HD, HD), :]
        acc = jnp.dot(
            ctx_ref[pl.ds(my * QR, QR)], wo_mine,
            preferred_element_type=jnp.float32,
        )
        for r in range(1, N_DEV):
            a_rdmas[r - 1].wait_recv()
            sender = lax.rem(my + N_DEV - r, N_DEV)
            wo_s = wo_ref[pl.ds(sender * HD, HD), :]
            acc = acc + jnp.dot(
                rs_ref[r], wo_s, preferred_element_type=jnp.float32)
        out_ref[pl.ds(my * QR, QR)] = acc

        b_rdmas = []
        for r in range(1, N_DEV):
            tgt = lax.rem(my + r, N_DEV)
            rdma = pltpu.make_async_remote_copy(
                src_ref=out_ref.at[pl.ds(my * QR, QR)],
                dst_ref=out_ref.at[pl.ds(my * QR, QR)],
                send_sem=ag_send.at[r],
                recv_sem=ag_recv.at[r],
                device_id=(tgt,),
                device_id_type=pl.DeviceIdType.MESH,
            )
            rdma.start()
            b_rdmas.append(rdma)
        for r in range(1, N_DEV):
            b_rdmas[r - 1].wait_recv()

        for r in range(1, N_DEV):
            a_rdmas[r - 1].wait_send()
            b_rdmas[r - 1].wait_send()

    out = pl.pallas_call(
        body,
        out_shape=jax.ShapeDtypeStruct((ROWS, D_MODEL), jnp.float32),
        in_specs=[pl.BlockSpec(memory_space=pltpu.VMEM)] * 5,
        out_specs=pl.BlockSpec(memory_space=pltpu.VMEM),
        scratch_shapes=[
            pltpu.VMEM((ROWS, HD), jnp.float32),
            pltpu.VMEM((N_DEV, QR, HD), jnp.float32),
            pltpu.SemaphoreType.DMA((N_DEV,)),
            pltpu.SemaphoreType.DMA((N_DEV,)),
            pltpu.SemaphoreType.DMA((N_DEV,)),
            pltpu.SemaphoreType.DMA((N_DEV,)),
        ],
        compiler_params=pltpu.CompilerParams(collective_id=0),
    )(x2, Wq, K2, V2, Wo)
    return out.reshape(B, SQ, D_MODEL)
